# baseline (speedup 1.0000x reference)
import functools

import jax
import jax.numpy as jnp
from jax import lax
from jax.experimental import pallas as pl
from jax.experimental.pallas import tpu as pltpu
from jax.experimental.pallas import tpu_sc as plsc

N, D = 16384, 512
TOTAL = N * D

NC, NS, L = 2, 16, 16
NW = NC * NS
PER_W = TOTAL // NW
CHUNK = 32 * D
NCHUNK = PER_W // CHUNK

_mesh = plsc.VectorSubcoreMesh(
    core_axis_name="c", subcore_axis_name="s", num_cores=NC, num_subcores=NS
)


@functools.partial(
    pl.kernel,
    out_type=jax.ShapeDtypeStruct((NW * L,), jnp.float32),
    mesh=_mesh,
    scratch_types=[
        pltpu.VMEM((2, CHUNK), jnp.float32),
        pltpu.VMEM((2, CHUNK), jnp.float32),
        pltpu.VMEM((L,), jnp.float32),
        pltpu.SemaphoreType.DMA,
        pltpu.SemaphoreType.DMA,
    ],
)
def _sc_sse(pred_hbm, true_hbm, out_hbm, pbuf, tbuf, accv, sem0, sem1):
    wid = lax.axis_index("s") * NC + lax.axis_index("c")
    base = wid * PER_W
    sems = (sem0, sem1)

    def start(k):
        slot = k % 2
        off = base + k * CHUNK
        cp = pltpu.async_copy(
            pred_hbm.at[pl.ds(off, CHUNK)], pbuf.at[slot], sems[slot]
        )
        ct = pltpu.async_copy(
            true_hbm.at[pl.ds(off, CHUNK)], tbuf.at[slot], sems[slot]
        )
        return cp, ct

    inflight = start(0)
    acc = jnp.zeros((L,), jnp.float32)
    for k in range(NCHUNK):
        slot = k % 2
        cp, ct = inflight
        cp.wait()
        ct.wait()
        if k + 1 < NCHUNK:
            inflight = start(k + 1)

        def body(j, a):
            d = pbuf[slot, pl.ds(j, L)] - tbuf[slot, pl.ds(j, L)]
            return a + d * d

        acc = plsc.parallel_loop(0, CHUNK, step=L, unroll=8, carry=acc)(body)

    accv[...] = acc
    pltpu.sync_copy(accv, out_hbm.at[pl.ds(wid * L, L)])


def kernel(y_pred, y_true):
    partials = _sc_sse(y_pred.reshape(-1), y_true.reshape(-1))
    return jnp.sum(partials) / N


# speedup vs baseline: 2.6576x; 2.6576x over previous
import functools

import jax
import jax.numpy as jnp
from jax import lax
from jax.experimental import pallas as pl
from jax.experimental.pallas import tpu as pltpu
from jax.experimental.pallas import tpu_sc as plsc

N, D = 16384, 512

NC, NS, L = 2, 16, 16
NW = NC * NS
ROWS_W = N // NW
CR = 32
NCHUNK = ROWS_W // CR
NACC = 8

_mesh = plsc.VectorSubcoreMesh(
    core_axis_name="c", subcore_axis_name="s", num_cores=NC, num_subcores=NS
)


@functools.partial(
    pl.kernel,
    out_type=jax.ShapeDtypeStruct((NW, L), jnp.float32),
    mesh=_mesh,
    scratch_types=[
        pltpu.VMEM((2, CR, D), jnp.float32),
        pltpu.VMEM((2, CR, D), jnp.float32),
        pltpu.VMEM((L,), jnp.float32),
        pltpu.SemaphoreType.DMA,
        pltpu.SemaphoreType.DMA,
    ],
    compiler_params=pltpu.CompilerParams(use_tc_tiling_on_sc=True),
)
def _sc_sse(pred_hbm, true_hbm, out_hbm, pbuf, tbuf, accv, sem0, sem1):
    wid = lax.axis_index("s") * NC + lax.axis_index("c")
    base = wid * ROWS_W
    sems = (sem0, sem1)

    def start(k):
        slot = k % 2
        row0 = base + k * CR
        cp = pltpu.async_copy(
            pred_hbm.at[pl.ds(row0, CR), :], pbuf.at[slot], sems[slot]
        )
        ct = pltpu.async_copy(
            true_hbm.at[pl.ds(row0, CR), :], tbuf.at[slot], sems[slot]
        )
        return cp, ct

    inflight = start(0)
    accs = tuple(jnp.zeros((L,), jnp.float32) for _ in range(NACC))
    for k in range(NCHUNK):
        slot = k % 2
        cp, ct = inflight
        cp.wait()
        ct.wait()
        if k + 1 < NCHUNK:
            inflight = start(k + 1)

        def body(r, a):
            a = list(a)
            for t in range(D // L):
                d = pbuf[slot, r, pl.ds(t * L, L)] - tbuf[slot, r, pl.ds(t * L, L)]
                a[t % NACC] = a[t % NACC] + d * d
            return tuple(a)

        accs = plsc.parallel_loop(0, CR, carry=accs)(body)

    acc = accs[0]
    for t in range(1, NACC):
        acc = acc + accs[t]
    accv[...] = acc
    pltpu.sync_copy(accv, out_hbm.at[wid])


def kernel(y_pred, y_true):
    partials = _sc_sse(y_pred, y_true)
    return jnp.sum(partials) / N


# speedup vs baseline: 3.9649x; 1.4919x over previous
import functools

import jax
import jax.numpy as jnp
from jax import lax
from jax.experimental import pallas as pl
from jax.experimental.pallas import tpu as pltpu
from jax.experimental.pallas import tpu_sc as plsc

N, D = 16384, 512

SC_ROWS = 4096
TC_ROWS = N - SC_ROWS

NC, NS, L = 2, 16, 16
NW = NC * NS
ROWS_W = SC_ROWS // NW
CR = 32
NCHUNK = ROWS_W // CR
NACC = 8

_mesh = plsc.VectorSubcoreMesh(
    core_axis_name="c", subcore_axis_name="s", num_cores=NC, num_subcores=NS
)


@functools.partial(
    pl.kernel,
    out_type=jax.ShapeDtypeStruct((NW, L), jnp.float32),
    mesh=_mesh,
    scratch_types=[
        pltpu.VMEM((2, CR, D), jnp.float32),
        pltpu.VMEM((2, CR, D), jnp.float32),
        pltpu.VMEM((L,), jnp.float32),
        pltpu.SemaphoreType.DMA,
        pltpu.SemaphoreType.DMA,
    ],
    compiler_params=pltpu.CompilerParams(use_tc_tiling_on_sc=True),
)
def _sc_sse(pred_hbm, true_hbm, out_hbm, pbuf, tbuf, accv, sem0, sem1):
    wid = lax.axis_index("s") * NC + lax.axis_index("c")
    base = wid * ROWS_W
    sems = (sem0, sem1)

    def start(k):
        slot = k % 2
        row0 = base + k * CR
        cp = pltpu.async_copy(
            pred_hbm.at[pl.ds(row0, CR), :], pbuf.at[slot], sems[slot]
        )
        ct = pltpu.async_copy(
            true_hbm.at[pl.ds(row0, CR), :], tbuf.at[slot], sems[slot]
        )
        return cp, ct

    inflight = start(0)
    accs = tuple(jnp.zeros((L,), jnp.float32) for _ in range(NACC))
    for k in range(NCHUNK):
        slot = k % 2
        cp, ct = inflight
        cp.wait()
        ct.wait()
        if k + 1 < NCHUNK:
            inflight = start(k + 1)

        def body(r, a):
            a = list(a)
            for t in range(D // L):
                d = pbuf[slot, r, pl.ds(t * L, L)] - tbuf[slot, r, pl.ds(t * L, L)]
                a[t % NACC] = a[t % NACC] + d * d
            return tuple(a)

        accs = plsc.parallel_loop(0, CR, carry=accs)(body)

    acc = accs[0]
    for t in range(1, NACC):
        acc = acc + accs[t]
    accv[...] = acc
    pltpu.sync_copy(accv, out_hbm.at[wid])


BLOCK_ROWS = 2048
TC_GRID = TC_ROWS // BLOCK_ROWS
TC_BLOCK0 = SC_ROWS // BLOCK_ROWS


def _tc_sse(pred_ref, true_ref, out_ref, acc_ref):
    i = pl.program_id(0)

    @pl.when(i == 0)
    def _():
        acc_ref[...] = jnp.zeros_like(acc_ref)

    d = pred_ref[...] - true_ref[...]
    acc_ref[...] += jnp.sum(d * d, axis=0, keepdims=True)

    @pl.when(i == TC_GRID - 1)
    def _():
        out_ref[...] = jnp.sum(acc_ref[...]).reshape(1, 1)


def kernel(y_pred, y_true):
    sc_partials = _sc_sse(y_pred, y_true)
    tc_total = pl.pallas_call(
        _tc_sse,
        grid=(TC_GRID,),
        in_specs=[
            pl.BlockSpec((BLOCK_ROWS, D), lambda i: (i + TC_BLOCK0, 0)),
            pl.BlockSpec((BLOCK_ROWS, D), lambda i: (i + TC_BLOCK0, 0)),
        ],
        out_specs=pl.BlockSpec((1, 1), lambda i: (0, 0)),
        out_shape=jax.ShapeDtypeStruct((1, 1), jnp.float32),
        scratch_shapes=[pltpu.VMEM((1, D), jnp.float32)],
    )(y_pred, y_true)
    return (jnp.sum(sc_partials) + tc_total[0, 0]) / N
